# baseline (speedup 1.0000x reference)
import functools

import jax
import jax.numpy as jnp
from jax import lax
from jax.experimental import pallas as pl
from jax.experimental.pallas import tpu as pltpu
from jax.experimental.pallas import tpu_sc as plsc

N = 16384
D = 32
K = 512
BLOCK_N = 2048

NC = 2
NS = 16
L = 16
NW = NC * NS
ROWS_PER_W = N // NW
CH = 128


def _dist_block(x_ref, c_ref, o_ref):
    x = x_ref[...]
    c = c_ref[...]
    s = lax.dot_general(c, x, (((0,), (1,)), ((), ())),
                        preferred_element_type=jnp.float32,
                        precision=lax.Precision.HIGHEST)
    cn = jnp.sum(c * c, axis=0)[:, None]
    o_ref[...] = cn - 2.0 * s


def _sc_argmin(dT_hbm, out_hbm, buf, outv):
    wid = lax.axis_index("s") * NC + lax.axis_index("c")
    base = wid * ROWS_PER_W
    for chunk in range(ROWS_PER_W // CH):
        col0 = base + chunk * CH
        pltpu.sync_copy(dT_hbm.at[:, pl.ds(col0, CH)], buf)

        def body(k, carry):
            kf = lax.convert_element_type(k, jnp.float32)
            new = []
            for g in range(CH // L):
                mv, mi = carry[2 * g], carry[2 * g + 1]
                v = buf[k, pl.ds(g * L, L)]
                p = v < mv
                new.append(jnp.where(p, v, mv))
                new.append(jnp.where(p, kf, mi))
            return tuple(new)

        init = []
        for g in range(CH // L):
            init.append(jnp.full((L,), jnp.inf, jnp.float32))
            init.append(jnp.zeros((L,), jnp.float32))
        res = lax.fori_loop(0, K, body, tuple(init))
        for g in range(CH // L):
            outv[pl.ds(chunk * CH + g * L, L)] = res[2 * g + 1]
    pltpu.sync_copy(outv, out_hbm.at[pl.ds(base, ROWS_PER_W)])


@jax.jit
def kernel(features, centroids):
    dT = pl.pallas_call(
        _dist_block,
        grid=(N // BLOCK_N,),
        in_specs=[
            pl.BlockSpec((BLOCK_N, D), lambda i: (i, 0)),
            pl.BlockSpec((D, K), lambda i: (0, 0)),
        ],
        out_specs=pl.BlockSpec((K, BLOCK_N), lambda i: (0, i)),
        out_shape=jax.ShapeDtypeStruct((K, N), jnp.float32),
    )(features, centroids)

    mesh = plsc.VectorSubcoreMesh(core_axis_name="c", subcore_axis_name="s",
                                  num_cores=NC, num_subcores=NS)
    out = pl.kernel(
        _sc_argmin,
        out_type=jax.ShapeDtypeStruct((N,), jnp.float32),
        mesh=mesh,
        scratch_types=[
            pltpu.VMEM((K, CH), jnp.float32),
            pltpu.VMEM((ROWS_PER_W,), jnp.float32),
        ],
    )(dT)
    return out[:, None]


# speedup vs baseline: 1.0998x; 1.0998x over previous
import functools

import jax
import jax.numpy as jnp
from jax import lax
from jax.experimental import pallas as pl
from jax.experimental.pallas import tpu as pltpu
from jax.experimental.pallas import tpu_sc as plsc

N = 16384
D = 32
K = 512
BLOCK_N = 2048

NC = 2
NS = 16
L = 16
NW = NC * NS
ROWS_PER_W = N // NW
CH = 128


def _dist_block(x_ref, c_ref, o_ref):
    x = x_ref[...]
    c = c_ref[...]
    s = lax.dot_general(c, x, (((0,), (1,)), ((), ())),
                        preferred_element_type=jnp.float32,
                        precision=lax.Precision.HIGHEST)
    cn = jnp.sum(c * c, axis=0)[:, None]
    o_ref[...] = cn - 2.0 * s


KH = 256
N_CHUNK = ROWS_PER_W // CH
N_PIECE = N_CHUNK * (K // KH)


def _sc_argmin(dT_hbm, out_hbm, bufs, outv, sems):
    wid = lax.axis_index("s") * NC + lax.axis_index("c")
    base = wid * ROWS_PER_W

    def start(i):
        chunk, half = divmod(i, K // KH)
        src = dT_hbm.at[pl.ds(half * KH, KH),
                        pl.ds(base + chunk * CH, CH)]
        return pltpu.async_copy(src, bufs.at[i % 2], sems.at[i % 2])

    copies = {0: start(0)}
    carry = None
    for i in range(N_PIECE):
        chunk, half = divmod(i, K // KH)
        if i + 1 < N_PIECE:
            copies[i + 1] = start(i + 1)
        copies.pop(i).wait()
        buf = bufs.at[i % 2]
        if half == 0:
            carry = []
            for g in range(CH // L):
                carry.append(jnp.full((L,), jnp.inf, jnp.float32))
                carry.append(jnp.zeros((L,), jnp.float32))
            carry = tuple(carry)

        def body(k, c, _half=half, _buf=buf):
            kf = lax.convert_element_type(k + _half * KH, jnp.float32)
            new = []
            for g in range(CH // L):
                mv, mi = c[2 * g], c[2 * g + 1]
                v = _buf[k, pl.ds(g * L, L)]
                p = v < mv
                new.append(jnp.where(p, v, mv))
                new.append(jnp.where(p, kf, mi))
            return tuple(new)

        carry = lax.fori_loop(0, KH, body, carry)
        if half == K // KH - 1:
            for g in range(CH // L):
                outv[pl.ds(chunk * CH + g * L, L)] = carry[2 * g + 1]
    pltpu.sync_copy(outv, out_hbm.at[pl.ds(base, ROWS_PER_W)])


@jax.jit
def kernel(features, centroids):
    dT = pl.pallas_call(
        _dist_block,
        grid=(N // BLOCK_N,),
        in_specs=[
            pl.BlockSpec((BLOCK_N, D), lambda i: (i, 0)),
            pl.BlockSpec((D, K), lambda i: (0, 0)),
        ],
        out_specs=pl.BlockSpec((K, BLOCK_N), lambda i: (0, i)),
        out_shape=jax.ShapeDtypeStruct((K, N), jnp.float32),
    )(features, centroids)

    mesh = plsc.VectorSubcoreMesh(core_axis_name="c", subcore_axis_name="s",
                                  num_cores=NC, num_subcores=NS)
    out = pl.kernel(
        _sc_argmin,
        out_type=jax.ShapeDtypeStruct((N,), jnp.float32),
        mesh=mesh,
        scratch_types=[
            pltpu.VMEM((2, KH, CH), jnp.float32),
            pltpu.VMEM((ROWS_PER_W,), jnp.float32),
            pltpu.SemaphoreType.DMA((2,)),
        ],
    )(dT)
    return out[:, None]
